# baseline (speedup 1.0000x reference)
import functools

import jax
import jax.numpy as jnp
from jax import lax
from jax.experimental import pallas as pl
from jax.experimental.pallas import tpu as pltpu
from jax.experimental.pallas import tpu_sc as plsc

VOCAB = 1000000
EMBED = 64
B = 4096
L = 200

_info = plsc.get_sparse_core_info()
NC, NS, LANES = _info.num_cores, _info.num_subcores, _info.num_lanes
NW = NC * NS
NBLK = VOCAB // 128
NPER = NBLK // NW


def _k1_transpose(src_v, dst_v):
    def diag8(rb, carry):
        for dr in range(8):
            r = 8 * rb + dr
            c_rot = lax.rem(lax.iota(jnp.int32, LANES) + r, LANES)
            for qt in range(8):
                t_l = c_rot + 16 * qt
                u_l = lax.shift_right_logical(t_l, 1)
                h_l = lax.shift_left(lax.bitwise_and(t_l, 1), 6)
                for qc in range(4):
                    c_l = lax.iota(jnp.int32, LANES) + 16 * qc
                    v = plsc.load_gather(src_v, [c_l, t_l])
                    plsc.store_scatter(dst_v, [u_l, h_l + c_l], v)
        return carry

    lax.fori_loop(0, 2, diag8, 0)


def _k1_body(tableT, tail128, scratch, src0, src1, src2, src3, dst0, dst1,
             si0, si1, si2, si3, so0, so1):
    wid = lax.axis_index("s") * NC + lax.axis_index("c")
    base = wid * NPER
    srcs, dsts = (src0, src1, src2, src3), (dst0, dst1)
    sis, sos = (si0, si1, si2, si3), (so0, so1)

    def in_copy(j, p):
        return pltpu.make_async_copy(
            tableT.at[:, pl.ds(j * 128, 128)], srcs[p], sis[p])

    def out_copy(j, p):
        return pltpu.make_async_copy(
            dsts[p], scratch.at[pl.ds(j * 64, 64), :], sos[p])

    for s in range(4):
        in_copy(base + s, s).start()

    def step(j, p4, p2):
        in_copy(j, p4).wait()

        @pl.when(j - base >= 2)
        def _():
            out_copy(j - 2, p2).wait()

        _k1_transpose(srcs[p4], dsts[p2])
        out_copy(j, p2).start()

        @pl.when(j + 4 < base + NPER)
        def _():
            in_copy(j + 4, p4).start()

    def body(i, carry):
        j = base + 4 * i
        step(j, 0, 0)
        step(j + 1, 1, 1)
        step(j + 2, 2, 0)
        step(j + 3, 3, 1)
        return carry

    lax.fori_loop(0, NPER // 4, body, 0)
    out_copy(base + NPER - 2, 0).wait()
    out_copy(base + NPER - 1, 1).wait()

    @pl.when(wid < NBLK - NW * NPER)
    def _tail_full():
        j = NW * NPER + wid
        in_copy(j, 0).start()
        in_copy(j, 0).wait()
        _k1_transpose(src0, dst0)
        out_copy(j, 0).start()
        out_copy(j, 0).wait()

    @pl.when(wid == NW - 1)
    def _tail_rem():
        pltpu.make_async_copy(tail128, src0, si0).start()
        pltpu.make_async_copy(tail128, src0, si0).wait()
        _k1_transpose(src0, dst0)
        cp = pltpu.make_async_copy(
            dst0, scratch.at[pl.ds(VOCAB // 2 - 64, 64), :], so0)
        cp.start()
        cp.wait()


def _k2_compute(rows_v, dst_v, pe_v, lm):
    u = lax.shift_right_logical(lm, 1)
    h = lax.shift_left(lax.bitwise_and(lm, 1), 6)
    lm_splat = jnp.full((LANES,), u, jnp.int32)

    def diag8(rb, carry):
        for dr in range(8):
            r = 8 * rb + dr
            c_rot = lax.rem(lax.iota(jnp.int32, LANES) + r, LANES)
            for qc in range(4):
                c_l = c_rot + 16 * qc
                pe_d = plsc.load_gather(pe_v, [lm_splat, c_l + h])
                for qt in range(8):
                    t_l = lax.iota(jnp.int32, LANES) + 16 * qt
                    v = plsc.load_gather(rows_v, [t_l, c_l])
                    plsc.store_scatter(dst_v, [c_l, t_l], v + pe_d)
        return carry

    lax.fori_loop(0, 2, diag8, 0)


def _k2_body(tokensT, scratch2, pe2, out, tok_v, rows0, rows1, rows2,
             dst0, dst1, dst2, pe_v, sg0, sg1, sg2, so0, so1, so2):
    wid = lax.axis_index("s") * NC + lax.axis_index("c")
    b0 = wid * 128
    rows, dsts = (rows0, rows1, rows2), (dst0, dst1, dst2)
    sgs, sos = (sg0, sg1, sg2), (so0, so1, so2)

    pltpu.sync_copy(pe2, pe_v)
    pltpu.sync_copy(tokensT.at[:, pl.ds(b0, 128)], tok_v)

    def gather_copy(l, p):
        return pltpu.make_async_copy(
            scratch2.at[tok_v.at[l]], rows[p], sgs[p])

    def out_copies(l, p):
        return [pltpu.make_async_copy(
            dsts[p].at[pl.ds(8 * a, 8), :], out.at[l, a, wid], sos[p])
            for a in range(8)]

    def arrive(l, p):
        gather_copy(l, p).start()

    def compute(lm, p):
        gather_copy(lm, p).wait()

        @pl.when(lm >= 3)
        def _():
            for cp in out_copies(lm - 3, p):
                cp.wait()

        _k2_compute(rows[p], dsts[p], pe_v, lm)
        for cp in out_copies(lm, p):
            cp.start()

    arrive(0, 0)
    arrive(1, 1)
    arrive(2, 2)

    def body(i, carry):
        l = 3 * i
        for s in range(3):
            compute(l + s, s)

            @pl.when(l + s + 3 < L)
            def _():
                arrive(l + s + 3, s)

        return carry

    lax.fori_loop(0, 66, body, 0)
    compute(198, 0)
    compute(199, 1)
    for lm, p in ((197, 2), (198, 0), (199, 1)):
        for cp in out_copies(lm, p):
            cp.wait()


@jax.jit
def kernel(tokens, table, pe):
    mesh = plsc.VectorSubcoreMesh(core_axis_name="c", subcore_axis_name="s")
    params = pltpu.CompilerParams(
        use_tc_tiling_on_sc=True, needs_layout_passes=False)

    k1 = functools.partial(
        pl.kernel, mesh=mesh,
        out_type=jax.ShapeDtypeStruct((VOCAB // 2, 128), jnp.float32),
        scratch_types=[
            pltpu.VMEM((EMBED, 128), jnp.float32),
            pltpu.VMEM((EMBED, 128), jnp.float32),
            pltpu.VMEM((EMBED, 128), jnp.float32),
            pltpu.VMEM((EMBED, 128), jnp.float32),
            pltpu.VMEM((EMBED, 128), jnp.float32),
            pltpu.VMEM((EMBED, 128), jnp.float32),
            pltpu.SemaphoreType.DMA,
            pltpu.SemaphoreType.DMA,
            pltpu.SemaphoreType.DMA,
            pltpu.SemaphoreType.DMA,
            pltpu.SemaphoreType.DMA,
            pltpu.SemaphoreType.DMA,
        ],
        compiler_params=params,
    )(_k1_body)
    tableT = table.T
    tail128 = lax.slice(tableT, (0, VOCAB - 128), (EMBED, VOCAB))
    scratch = k1(tableT, tail128)

    params2 = pltpu.CompilerParams(
        use_tc_tiling_on_sc=False, needs_layout_passes=False)
    k2 = functools.partial(
        pl.kernel, mesh=mesh,
        out_type=jax.ShapeDtypeStruct((L, 8, NW, 8, 128), jnp.float32),
        scratch_types=[
            pltpu.VMEM((L, 128), jnp.int32),
            pltpu.VMEM((128, EMBED), jnp.float32),
            pltpu.VMEM((128, EMBED), jnp.float32),
            pltpu.VMEM((128, EMBED), jnp.float32),
            pltpu.VMEM((EMBED, 128), jnp.float32),
            pltpu.VMEM((EMBED, 128), jnp.float32),
            pltpu.VMEM((EMBED, 128), jnp.float32),
            pltpu.VMEM((L // 2, 128), jnp.float32),
            pltpu.SemaphoreType.DMA,
            pltpu.SemaphoreType.DMA,
            pltpu.SemaphoreType.DMA,
            pltpu.SemaphoreType.DMA,
            pltpu.SemaphoreType.DMA,
            pltpu.SemaphoreType.DMA,
        ],
        compiler_params=params2,
    )(_k2_body)
    scratch2 = scratch.reshape(VOCAB, EMBED)
    out5 = k2(tokens.T, scratch2, pe.reshape(L // 2, 128))
    return out5.transpose(2, 4, 0, 1, 3).reshape(B, L, EMBED)


# speedup vs baseline: 1.7621x; 1.7621x over previous
import functools

import jax
import jax.numpy as jnp
from jax import lax
from jax.experimental import pallas as pl
from jax.experimental.pallas import tpu as pltpu
from jax.experimental.pallas import tpu_sc as plsc

VOCAB = 1000000
EMBED = 64
B = 4096
L = 200

_info = plsc.get_sparse_core_info()
NC, NS, LANES = _info.num_cores, _info.num_subcores, _info.num_lanes
NW = NC * NS
NBLK = VOCAB // 128
NPER = NBLK // NW


def _k1_transpose(src_v, dst_v):
    def diag4(rb, carry):
        for dr in range(4):
            r = 4 * rb + dr
            c_rot = lax.rem(lax.iota(jnp.int32, LANES) + r, LANES)
            for qt in range(8):
                t_l = c_rot + 16 * qt
                u_l = lax.shift_right_logical(t_l, 1)
                h_l = lax.shift_left(lax.bitwise_and(t_l, 1), 6)
                for qc in range(4):
                    c_l = lax.iota(jnp.int32, LANES) + 16 * qc
                    v = plsc.load_gather(src_v, [c_l, t_l])
                    plsc.store_scatter(dst_v, [u_l, h_l + c_l], v)
        return carry

    lax.fori_loop(0, 4, diag4, 0)


def _k1_body(tableT, tail128, scratch, src0, src1, src2, src3, dst0, dst1,
             si0, si1, si2, si3, so0, so1):
    wid = lax.axis_index("s") * NC + lax.axis_index("c")
    base = wid * NPER
    srcs, dsts = (src0, src1, src2, src3), (dst0, dst1)
    sis, sos = (si0, si1, si2, si3), (so0, so1)

    def in_copy(j, p):
        return pltpu.make_async_copy(
            tableT.at[:, pl.ds(j * 128, 128)], srcs[p], sis[p])

    def out_copy(j, p):
        return pltpu.make_async_copy(
            dsts[p], scratch.at[pl.ds(j * 64, 64), :], sos[p])

    for s in range(4):
        in_copy(base + s, s).start()

    def step(j, p4, p2):
        in_copy(j, p4).wait()

        @pl.when(j - base >= 2)
        def _():
            out_copy(j - 2, p2).wait()

        _k1_transpose(srcs[p4], dsts[p2])
        out_copy(j, p2).start()

        @pl.when(j + 4 < base + NPER)
        def _():
            in_copy(j + 4, p4).start()

    def body(i, carry):
        j = base + 4 * i
        step(j, 0, 0)
        step(j + 1, 1, 1)
        step(j + 2, 2, 0)
        step(j + 3, 3, 1)
        return carry

    lax.fori_loop(0, NPER // 4, body, 0)
    out_copy(base + NPER - 2, 0).wait()
    out_copy(base + NPER - 1, 1).wait()

    @pl.when(wid < NBLK - NW * NPER)
    def _tail_full():
        j = NW * NPER + wid
        in_copy(j, 0).start()
        in_copy(j, 0).wait()
        _k1_transpose(src0, dst0)
        out_copy(j, 0).start()
        out_copy(j, 0).wait()

    @pl.when(wid == NW - 1)
    def _tail_rem():
        pltpu.make_async_copy(tail128, src0, si0).start()
        pltpu.make_async_copy(tail128, src0, si0).wait()
        _k1_transpose(src0, dst0)
        cp = pltpu.make_async_copy(
            dst0, scratch.at[pl.ds(VOCAB // 2 - 64, 64), :], so0)
        cp.start()
        cp.wait()


def _k2_compute(rows_v, dst_v, pe_v, lm):
    u = lax.shift_right_logical(lm, 1)
    h = lax.shift_left(lax.bitwise_and(lm, 1), 6)
    lm_splat = jnp.full((LANES,), u, jnp.int32)

    def diag4(rb, carry):
        for dr in range(4):
            r = 4 * rb + dr
            c_rot = lax.rem(lax.iota(jnp.int32, LANES) + r, LANES)
            for qc in range(4):
                c_l = c_rot + 16 * qc
                pe_d = plsc.load_gather(pe_v, [lm_splat, c_l + h])
                for qt in range(8):
                    t_l = lax.iota(jnp.int32, LANES) + 16 * qt
                    v = plsc.load_gather(rows_v, [t_l, c_l])
                    plsc.store_scatter(dst_v, [c_l, t_l], v + pe_d)
        return carry

    lax.fori_loop(0, 4, diag4, 0)


def _k2_body(tokensT, scratch2, pe2, out, tok_v, rows0, rows1, rows2,
             dst0, dst1, dst2, pe_v, sg0, sg1, sg2, so0, so1, so2):
    wid = lax.axis_index("s") * NC + lax.axis_index("c")
    b0 = wid * 128
    rows, dsts = (rows0, rows1, rows2), (dst0, dst1, dst2)
    sgs, sos = (sg0, sg1, sg2), (so0, so1, so2)

    pltpu.sync_copy(pe2, pe_v)
    pltpu.sync_copy(tokensT.at[:, pl.ds(b0, 128)], tok_v)

    def gather_copy(l, p):
        return pltpu.make_async_copy(
            scratch2.at[tok_v.at[l]], rows[p], sgs[p])

    def out_copies(l, p):
        return [pltpu.make_async_copy(
            dsts[p].at[pl.ds(8 * a, 8), :], out.at[l, a, wid], sos[p])
            for a in range(8)]

    def arrive(l, p):
        gather_copy(l, p).start()

    def compute(lm, p):
        gather_copy(lm, p).wait()

        @pl.when(lm >= 3)
        def _():
            for cp in out_copies(lm - 3, p):
                cp.wait()

        _k2_compute(rows[p], dsts[p], pe_v, lm)
        for cp in out_copies(lm, p):
            cp.start()

    arrive(0, 0)
    arrive(1, 1)
    arrive(2, 2)

    def body(i, carry):
        l = 3 * i
        for s in range(3):
            compute(l + s, s)

            @pl.when(l + s + 3 < L)
            def _():
                arrive(l + s + 3, s)

        return carry

    lax.fori_loop(0, 66, body, 0)
    compute(198, 0)
    compute(199, 1)
    for lm, p in ((197, 2), (198, 0), (199, 1)):
        for cp in out_copies(lm, p):
            cp.wait()


@jax.jit
def kernel(tokens, table, pe):
    mesh = plsc.VectorSubcoreMesh(core_axis_name="c", subcore_axis_name="s")
    params = pltpu.CompilerParams(
        use_tc_tiling_on_sc=True, needs_layout_passes=False)

    k1 = functools.partial(
        pl.kernel, mesh=mesh,
        out_type=jax.ShapeDtypeStruct((VOCAB // 2, 128), jnp.float32),
        scratch_types=[
            pltpu.VMEM((EMBED, 128), jnp.float32),
            pltpu.VMEM((EMBED, 128), jnp.float32),
            pltpu.VMEM((EMBED, 128), jnp.float32),
            pltpu.VMEM((EMBED, 128), jnp.float32),
            pltpu.VMEM((EMBED, 128), jnp.float32),
            pltpu.VMEM((EMBED, 128), jnp.float32),
            pltpu.SemaphoreType.DMA,
            pltpu.SemaphoreType.DMA,
            pltpu.SemaphoreType.DMA,
            pltpu.SemaphoreType.DMA,
            pltpu.SemaphoreType.DMA,
            pltpu.SemaphoreType.DMA,
        ],
        compiler_params=params,
    )(_k1_body)
    tableT = table.T
    tail128 = lax.slice(tableT, (0, VOCAB - 128), (EMBED, VOCAB))
    scratch = k1(tableT, tail128)

    params2 = pltpu.CompilerParams(
        use_tc_tiling_on_sc=False, needs_layout_passes=False)
    k2 = functools.partial(
        pl.kernel, mesh=mesh,
        out_type=jax.ShapeDtypeStruct((L, 8, NW, 8, 128), jnp.float32),
        scratch_types=[
            pltpu.VMEM((L, 128), jnp.int32),
            pltpu.VMEM((128, EMBED), jnp.float32),
            pltpu.VMEM((128, EMBED), jnp.float32),
            pltpu.VMEM((128, EMBED), jnp.float32),
            pltpu.VMEM((EMBED, 128), jnp.float32),
            pltpu.VMEM((EMBED, 128), jnp.float32),
            pltpu.VMEM((EMBED, 128), jnp.float32),
            pltpu.VMEM((L // 2, 128), jnp.float32),
            pltpu.SemaphoreType.DMA,
            pltpu.SemaphoreType.DMA,
            pltpu.SemaphoreType.DMA,
            pltpu.SemaphoreType.DMA,
            pltpu.SemaphoreType.DMA,
            pltpu.SemaphoreType.DMA,
        ],
        compiler_params=params2,
    )(_k2_body)
    scratch2 = scratch.reshape(VOCAB, EMBED)
    out5 = k2(tokens.T, scratch2, pe.reshape(L // 2, 128))
    return out5.transpose(2, 4, 0, 1, 3).reshape(B, L, EMBED)
